# baseline (speedup 1.0000x reference)
import functools

import jax
import jax.numpy as jnp
from jax import lax
from jax.experimental import pallas as pl
from jax.experimental.pallas import tpu as pltpu
from jax.experimental.pallas import tpu_sc as plsc

N = 10000
D = 128
E = 320000
NC = 2
NS = 16
NW = NC * NS
CH = 128
EW = E // NW
KF = EW // CH
TAIL = EW - KF * CH
N_PAD = 10240
RT = N_PAD // NS

def _sc_deg_body(dst_hbm, ones_hbm, z1_hbm, deg_out, deg_sh,
                 dstv0, dstv1, dtail, onesv, semd0, semd1):
    cid = lax.axis_index("c")
    sid = lax.axis_index("s")
    w = cid * NS + sid
    pltpu.sync_copy(z1_hbm, deg_sh.at[pl.ds(sid * RT, RT)])
    pltpu.sync_copy(ones_hbm, onesv)
    plsc.subcore_barrier()
    base0 = w * EW

    def dld(j, dstv, semd):
        return pltpu.async_copy(dst_hbm.at[pl.ds(base0 + j * CH, CH)],
                                dstv, semd)

    def retire(j, dstv, semd):
        pltpu.make_async_copy(dst_hbm.at[pl.ds(base0 + j * CH, CH)],
                              dstv, semd).wait()
        pltpu.sync_copy(onesv, deg_sh.at[dstv], add=True)

    dld(0, dstv0, semd0)

    def body(i, c):
        j = 2 * i
        dld(j + 1, dstv1, semd1)
        retire(j, dstv0, semd0)

        @pl.when(j + 2 < KF)
        def _():
            dld(j + 2, dstv0, semd0)

        retire(j + 1, dstv1, semd1)
        return c

    lax.fori_loop(0, KF // 2, body, 0)
    pltpu.sync_copy(dst_hbm.at[pl.ds(base0 + KF * CH, TAIL)], dtail)
    pltpu.sync_copy(onesv.at[pl.ds(0, TAIL)], deg_sh.at[dtail], add=True)
    plsc.subcore_barrier()
    pltpu.sync_copy(deg_sh.at[pl.ds(sid * RT, RT)],
                    deg_out.at[cid, pl.ds(sid * RT, RT)])


@functools.cache
def _get_sc_deg():
    mesh = plsc.VectorSubcoreMesh(core_axis_name="c", subcore_axis_name="s")
    return pl.kernel(
        _sc_deg_body,
        out_type=jax.ShapeDtypeStruct((NC, N_PAD), jnp.float32),
        mesh=mesh,
        scratch_types=[
            pltpu.VMEM_SHARED((N_PAD,), jnp.float32),
            pltpu.VMEM((CH,), jnp.int32),
            pltpu.VMEM((CH,), jnp.int32),
            pltpu.VMEM((TAIL,), jnp.int32),
            pltpu.VMEM((CH,), jnp.float32),
            pltpu.SemaphoreType.DMA,
            pltpu.SemaphoreType.DMA,
        ],
    )


def _sc_edge_body(y_hbm, src_hbm, dst_hbm, zr_hbm, acc_out,
                  acc_sh, srcv, dstv0, dstv1, dtail, rows0, rows1,
                  sem0, sem1, semd0, semd1):
    cid = lax.axis_index("c")
    sid = lax.axis_index("s")
    w = cid * NS + sid
    pltpu.sync_copy(zr_hbm, acc_sh.at[pl.ds(sid * RT, RT)])
    pltpu.sync_copy(src_hbm.at[pl.ds(w * EW, EW)], srcv)
    plsc.subcore_barrier()
    base0 = w * EW

    def g(j, rows, sem):
        return pltpu.async_copy(y_hbm.at[srcv.at[pl.ds(j * CH, CH)]],
                                rows, sem)

    def dld(j, dstv, semd):
        return pltpu.async_copy(dst_hbm.at[pl.ds(base0 + j * CH, CH)],
                                dstv, semd)

    def retire(j, rows, sem, dstv, semd):
        pltpu.make_async_copy(y_hbm.at[srcv.at[pl.ds(j * CH, CH)]],
                              rows, sem).wait()
        pltpu.make_async_copy(dst_hbm.at[pl.ds(base0 + j * CH, CH)],
                              dstv, semd).wait()
        pltpu.sync_copy(rows, acc_sh.at[dstv], add=True)

    dld(0, dstv0, semd0)
    g(0, rows0, sem0)

    def body(i, c):
        j = 2 * i
        dld(j + 1, dstv1, semd1)
        g(j + 1, rows1, sem1)
        retire(j, rows0, sem0, dstv0, semd0)

        @pl.when(j + 2 < KF)
        def _():
            dld(j + 2, dstv0, semd0)
            g(j + 2, rows0, sem0)

        retire(j + 1, rows1, sem1, dstv1, semd1)
        return c

    lax.fori_loop(0, KF // 2, body, 0)
    pltpu.sync_copy(dst_hbm.at[pl.ds(base0 + KF * CH, TAIL)], dtail)
    pltpu.sync_copy(y_hbm.at[srcv.at[pl.ds(KF * CH, TAIL)]],
                    rows0.at[pl.ds(0, TAIL)])
    pltpu.sync_copy(rows0.at[pl.ds(0, TAIL)], acc_sh.at[dtail], add=True)

    plsc.subcore_barrier()
    pltpu.sync_copy(acc_sh.at[pl.ds(sid * RT, RT)],
                    acc_out.at[cid, pl.ds(sid * RT, RT)])


@functools.cache
def _get_sc_edge():
    mesh = plsc.VectorSubcoreMesh(core_axis_name="c", subcore_axis_name="s")
    return pl.kernel(
        _sc_edge_body,
        out_type=jax.ShapeDtypeStruct((NC, N_PAD, D), jnp.float32),
        mesh=mesh,
        scratch_types=[
            pltpu.VMEM_SHARED((N_PAD, D), jnp.float32),
            pltpu.VMEM((EW,), jnp.int32),
            pltpu.VMEM((CH,), jnp.int32),
            pltpu.VMEM((CH,), jnp.int32),
            pltpu.VMEM((TAIL,), jnp.int32),
            pltpu.VMEM((CH, D), jnp.float32),
            pltpu.VMEM((CH, D), jnp.float32),
            pltpu.SemaphoreType.DMA,
            pltpu.SemaphoreType.DMA,
            pltpu.SemaphoreType.DMA,
            pltpu.SemaphoreType.DMA,
        ],
    )


_BN = 1000


def _tc_y1_body(d0_ref, d1_ref, x_ref, w_ref, y_ref, dinv_ref):
    deg = d0_ref[...] + d1_ref[...] + 1.0
    dinv = lax.rsqrt(deg)
    xw = jnp.dot(x_ref[...], w_ref[...], preferred_element_type=jnp.float32)
    y_ref[...] = dinv * xw
    dinv_ref[...] = dinv


def _tc_y1(d0, d1, x, w):
    return pl.pallas_call(
        _tc_y1_body,
        grid=(N // _BN,),
        in_specs=[
            pl.BlockSpec((_BN, 1), lambda i: (i, 0)),
            pl.BlockSpec((_BN, 1), lambda i: (i, 0)),
            pl.BlockSpec((_BN, D), lambda i: (i, 0)),
            pl.BlockSpec((D, D), lambda i: (0, 0)),
        ],
        out_specs=[
            pl.BlockSpec((_BN, D), lambda i: (i, 0)),
            pl.BlockSpec((_BN, 1), lambda i: (i, 0)),
        ],
        out_shape=[
            jax.ShapeDtypeStruct((N, D), jnp.float32),
            jax.ShapeDtypeStruct((N, 1), jnp.float32),
        ],
    )(d0, d1, x, w)


def _tc_mid_body(a_ref, y_ref, dinv_ref, b_ref, w_ref, y2_ref):
    dinv = dinv_ref[...]
    h = b_ref[...] + dinv * (a_ref[0] + a_ref[1] + y_ref[...])
    h = jnp.maximum(h, 0.0)
    y2_ref[...] = dinv * jnp.dot(h, w_ref[...],
                                 preferred_element_type=jnp.float32)


def _tc_mid(accp, y1, dinv, b, w):
    return pl.pallas_call(
        _tc_mid_body,
        grid=(N // _BN,),
        in_specs=[
            pl.BlockSpec((NC, _BN, D), lambda i: (0, i, 0)),
            pl.BlockSpec((_BN, D), lambda i: (i, 0)),
            pl.BlockSpec((_BN, 1), lambda i: (i, 0)),
            pl.BlockSpec((1, D), lambda i: (0, 0)),
            pl.BlockSpec((D, D), lambda i: (0, 0)),
        ],
        out_specs=pl.BlockSpec((_BN, D), lambda i: (i, 0)),
        out_shape=jax.ShapeDtypeStruct((N, D), jnp.float32),
    )(accp, y1, dinv, b, w)


def _tc_final_body(a_ref, y_ref, dinv_ref, b_ref, wfc_ref, bfc_ref,
                   out_ref, g_ref):
    i = pl.program_id(0)
    h = b_ref[...] + dinv_ref[...] * (a_ref[0] + a_ref[1] + y_ref[...])
    h = jnp.maximum(h, 0.0)

    @pl.when(i == 0)
    def _():
        g_ref[...] = jnp.zeros_like(g_ref)

    g_ref[...] += jnp.sum(h, axis=0, keepdims=True)

    @pl.when(i == N // _BN - 1)
    def _():
        g = g_ref[...] * (1.0 / N)
        out_ref[...] = jnp.dot(g, wfc_ref[...],
                               preferred_element_type=jnp.float32) + bfc_ref[...]


def _tc_final(accp, y2, dinv, b, wfc, bfc):
    return pl.pallas_call(
        _tc_final_body,
        grid=(N // _BN,),
        in_specs=[
            pl.BlockSpec((NC, _BN, D), lambda i: (0, i, 0)),
            pl.BlockSpec((_BN, D), lambda i: (i, 0)),
            pl.BlockSpec((_BN, 1), lambda i: (i, 0)),
            pl.BlockSpec((1, D), lambda i: (0, 0)),
            pl.BlockSpec((D, D), lambda i: (0, 0)),
            pl.BlockSpec((1, D), lambda i: (0, 0)),
        ],
        out_specs=pl.BlockSpec((1, D), lambda i: (0, 0)),
        out_shape=jax.ShapeDtypeStruct((1, D), jnp.float32),
        scratch_shapes=[pltpu.VMEM((1, D), jnp.float32)],
    )(accp, y2, dinv, b, wfc, bfc)


def kernel(x, edge_index, W1, b1, W2, b2, Wfc, bfc):
    src = edge_index[0]
    dst = edge_index[1]
    ones_ch = jnp.ones((CH,), jnp.float32)
    z1 = jnp.zeros((RT,), jnp.float32)
    zr = jnp.zeros((RT, D), jnp.float32)

    degp = _get_sc_deg()(dst, ones_ch, z1)
    d0 = degp[0, :N].reshape(N, 1)
    d1 = degp[1, :N].reshape(N, 1)

    y1, dinv = _tc_y1(d0, d1, x, W1)

    accp = _get_sc_edge()(y1, src, dst, zr)
    y2 = _tc_mid(accp, y1, dinv, b1.reshape(1, D), W2)

    accp2 = _get_sc_edge()(y2, src, dst, zr)
    out = _tc_final(accp2, y2, dinv, b2.reshape(1, D), Wfc, bfc.reshape(1, D))
    return out.reshape(D)


# speedup vs baseline: 1.0562x; 1.0562x over previous
import functools

import jax
import jax.numpy as jnp
from jax import lax
from jax.experimental import pallas as pl
from jax.experimental.pallas import tpu as pltpu
from jax.experimental.pallas import tpu_sc as plsc

N = 10000
D = 128
E = 320000
NC = 2
NS = 16
NW = NC * NS
CH = 128
EW = E // NW
KF = EW // CH
TAIL = EW - KF * CH
N_PAD = 10240
RT = N_PAD // NS

def _sc_deg_body(dst_hbm, ones_hbm, z1_hbm, deg_out, deg_sh,
                 dstv, onesv, semA, semB):
    cid = lax.axis_index("c")
    sid = lax.axis_index("s")
    w = cid * NS + sid
    pltpu.sync_copy(z1_hbm, deg_sh.at[pl.ds(sid * RT, RT)])
    pltpu.sync_copy(ones_hbm, onesv)
    pltpu.sync_copy(dst_hbm.at[pl.ds(w * EW, EW)], dstv)
    plsc.subcore_barrier()

    def sc(j, sem):
        return pltpu.async_copy(
            onesv, deg_sh.at[dstv.at[pl.ds(j * CH, CH)]], sem, add=True)

    def scw(j, sem):
        pltpu.make_async_copy(
            onesv, deg_sh.at[dstv.at[pl.ds(j * CH, CH)]], sem).wait()

    sc(0, semA)

    def body(i, c):
        j = 2 * i
        sc(j + 1, semB)
        scw(j, semA)

        @pl.when(j + 2 < KF)
        def _():
            sc(j + 2, semA)

        scw(j + 1, semB)
        return c

    lax.fori_loop(0, KF // 2, body, 0)
    pltpu.sync_copy(onesv.at[pl.ds(0, TAIL)],
                    deg_sh.at[dstv.at[pl.ds(KF * CH, TAIL)]], add=True)
    plsc.subcore_barrier()
    pltpu.sync_copy(deg_sh.at[pl.ds(sid * RT, RT)],
                    deg_out.at[cid, pl.ds(sid * RT, RT)])


@functools.cache
def _get_sc_deg():
    mesh = plsc.VectorSubcoreMesh(core_axis_name="c", subcore_axis_name="s")
    return pl.kernel(
        _sc_deg_body,
        out_type=jax.ShapeDtypeStruct((NC, N_PAD), jnp.float32),
        mesh=mesh,
        scratch_types=[
            pltpu.VMEM_SHARED((N_PAD,), jnp.float32),
            pltpu.VMEM((EW,), jnp.int32),
            pltpu.VMEM((CH,), jnp.float32),
            pltpu.SemaphoreType.DMA,
            pltpu.SemaphoreType.DMA,
        ],
    )


def _sc_edge_body(y_hbm, src_hbm, dst_hbm, zr_hbm, acc_out,
                  acc_sh, srcv, dstv0, dstv1, dtail, rows0, rows1,
                  sem0, sem1, semd0, semd1):
    cid = lax.axis_index("c")
    sid = lax.axis_index("s")
    w = cid * NS + sid
    pltpu.sync_copy(zr_hbm, acc_sh.at[pl.ds(sid * RT, RT)])
    pltpu.sync_copy(src_hbm.at[pl.ds(w * EW, EW)], srcv)
    plsc.subcore_barrier()
    base0 = w * EW

    def g(j, rows, sem):
        return pltpu.async_copy(y_hbm.at[srcv.at[pl.ds(j * CH, CH)]],
                                rows, sem)

    def dld(j, dstv, semd):
        return pltpu.async_copy(dst_hbm.at[pl.ds(base0 + j * CH, CH)],
                                dstv, semd)

    def retire(j, rows, sem, dstv, semd):
        pltpu.make_async_copy(y_hbm.at[srcv.at[pl.ds(j * CH, CH)]],
                              rows, sem).wait()
        pltpu.make_async_copy(dst_hbm.at[pl.ds(base0 + j * CH, CH)],
                              dstv, semd).wait()
        pltpu.sync_copy(rows, acc_sh.at[dstv], add=True)

    dld(0, dstv0, semd0)
    g(0, rows0, sem0)

    def body(i, c):
        j = 2 * i
        dld(j + 1, dstv1, semd1)
        g(j + 1, rows1, sem1)
        retire(j, rows0, sem0, dstv0, semd0)

        @pl.when(j + 2 < KF)
        def _():
            dld(j + 2, dstv0, semd0)
            g(j + 2, rows0, sem0)

        retire(j + 1, rows1, sem1, dstv1, semd1)
        return c

    lax.fori_loop(0, KF // 2, body, 0)
    pltpu.sync_copy(dst_hbm.at[pl.ds(base0 + KF * CH, TAIL)], dtail)
    pltpu.sync_copy(y_hbm.at[srcv.at[pl.ds(KF * CH, TAIL)]],
                    rows0.at[pl.ds(0, TAIL)])
    pltpu.sync_copy(rows0.at[pl.ds(0, TAIL)], acc_sh.at[dtail], add=True)

    plsc.subcore_barrier()
    pltpu.sync_copy(acc_sh.at[pl.ds(sid * RT, RT)],
                    acc_out.at[cid, pl.ds(sid * RT, RT)])


@functools.cache
def _get_sc_edge():
    mesh = plsc.VectorSubcoreMesh(core_axis_name="c", subcore_axis_name="s")
    return pl.kernel(
        _sc_edge_body,
        out_type=jax.ShapeDtypeStruct((NC, N_PAD, D), jnp.float32),
        mesh=mesh,
        scratch_types=[
            pltpu.VMEM_SHARED((N_PAD, D), jnp.float32),
            pltpu.VMEM((EW,), jnp.int32),
            pltpu.VMEM((CH,), jnp.int32),
            pltpu.VMEM((CH,), jnp.int32),
            pltpu.VMEM((TAIL,), jnp.int32),
            pltpu.VMEM((CH, D), jnp.float32),
            pltpu.VMEM((CH, D), jnp.float32),
            pltpu.SemaphoreType.DMA,
            pltpu.SemaphoreType.DMA,
            pltpu.SemaphoreType.DMA,
            pltpu.SemaphoreType.DMA,
        ],
    )


_BN = 1000


def _tc_y1_body(d0_ref, d1_ref, x_ref, w_ref, y_ref, dinv_ref):
    deg = d0_ref[...] + d1_ref[...] + 1.0
    dinv = lax.rsqrt(deg)
    xw = jnp.dot(x_ref[...], w_ref[...], preferred_element_type=jnp.float32)
    y_ref[...] = dinv * xw
    dinv_ref[...] = dinv


def _tc_y1(d0, d1, x, w):
    return pl.pallas_call(
        _tc_y1_body,
        grid=(N // _BN,),
        in_specs=[
            pl.BlockSpec((_BN, 1), lambda i: (i, 0)),
            pl.BlockSpec((_BN, 1), lambda i: (i, 0)),
            pl.BlockSpec((_BN, D), lambda i: (i, 0)),
            pl.BlockSpec((D, D), lambda i: (0, 0)),
        ],
        out_specs=[
            pl.BlockSpec((_BN, D), lambda i: (i, 0)),
            pl.BlockSpec((_BN, 1), lambda i: (i, 0)),
        ],
        out_shape=[
            jax.ShapeDtypeStruct((N, D), jnp.float32),
            jax.ShapeDtypeStruct((N, 1), jnp.float32),
        ],
    )(d0, d1, x, w)


def _tc_mid_body(a_ref, y_ref, dinv_ref, b_ref, w_ref, y2_ref):
    dinv = dinv_ref[...]
    h = b_ref[...] + dinv * (a_ref[0] + a_ref[1] + y_ref[...])
    h = jnp.maximum(h, 0.0)
    y2_ref[...] = dinv * jnp.dot(h, w_ref[...],
                                 preferred_element_type=jnp.float32)


def _tc_mid(accp, y1, dinv, b, w):
    return pl.pallas_call(
        _tc_mid_body,
        grid=(N // _BN,),
        in_specs=[
            pl.BlockSpec((NC, _BN, D), lambda i: (0, i, 0)),
            pl.BlockSpec((_BN, D), lambda i: (i, 0)),
            pl.BlockSpec((_BN, 1), lambda i: (i, 0)),
            pl.BlockSpec((1, D), lambda i: (0, 0)),
            pl.BlockSpec((D, D), lambda i: (0, 0)),
        ],
        out_specs=pl.BlockSpec((_BN, D), lambda i: (i, 0)),
        out_shape=jax.ShapeDtypeStruct((N, D), jnp.float32),
    )(accp, y1, dinv, b, w)


def _tc_final_body(a_ref, y_ref, dinv_ref, b_ref, wfc_ref, bfc_ref,
                   out_ref, g_ref):
    i = pl.program_id(0)
    h = b_ref[...] + dinv_ref[...] * (a_ref[0] + a_ref[1] + y_ref[...])
    h = jnp.maximum(h, 0.0)

    @pl.when(i == 0)
    def _():
        g_ref[...] = jnp.zeros_like(g_ref)

    g_ref[...] += jnp.sum(h, axis=0, keepdims=True)

    @pl.when(i == N // _BN - 1)
    def _():
        g = g_ref[...] * (1.0 / N)
        out_ref[...] = jnp.dot(g, wfc_ref[...],
                               preferred_element_type=jnp.float32) + bfc_ref[...]


def _tc_final(accp, y2, dinv, b, wfc, bfc):
    return pl.pallas_call(
        _tc_final_body,
        grid=(N // _BN,),
        in_specs=[
            pl.BlockSpec((NC, _BN, D), lambda i: (0, i, 0)),
            pl.BlockSpec((_BN, D), lambda i: (i, 0)),
            pl.BlockSpec((_BN, 1), lambda i: (i, 0)),
            pl.BlockSpec((1, D), lambda i: (0, 0)),
            pl.BlockSpec((D, D), lambda i: (0, 0)),
            pl.BlockSpec((1, D), lambda i: (0, 0)),
        ],
        out_specs=pl.BlockSpec((1, D), lambda i: (0, 0)),
        out_shape=jax.ShapeDtypeStruct((1, D), jnp.float32),
        scratch_shapes=[pltpu.VMEM((1, D), jnp.float32)],
    )(accp, y2, dinv, b, wfc, bfc)


def kernel(x, edge_index, W1, b1, W2, b2, Wfc, bfc):
    src = edge_index[0]
    dst = edge_index[1]
    ones_ch = jnp.ones((CH,), jnp.float32)
    z1 = jnp.zeros((RT,), jnp.float32)
    zr = jnp.zeros((RT, D), jnp.float32)

    degp = _get_sc_deg()(dst, ones_ch, z1)
    d0 = degp[0, :N].reshape(N, 1)
    d1 = degp[1, :N].reshape(N, 1)

    y1, dinv = _tc_y1(d0, d1, x, W1)

    accp = _get_sc_edge()(y1, src, dst, zr)
    y2 = _tc_mid(accp, y1, dinv, b1.reshape(1, D), W2)

    accp2 = _get_sc_edge()(y2, src, dst, zr)
    out = _tc_final(accp2, y2, dinv, b2.reshape(1, D), Wfc, bfc.reshape(1, D))
    return out.reshape(D)


# speedup vs baseline: 1.0977x; 1.0393x over previous
import functools

import jax
import jax.numpy as jnp
from jax import lax
from jax.experimental import pallas as pl
from jax.experimental.pallas import tpu as pltpu
from jax.experimental.pallas import tpu_sc as plsc

N = 10000
D = 128
E = 320000
NC = 2
NS = 16
NW = NC * NS
CH = 128
EW = E // NW
KF = EW // CH
TAIL = EW - KF * CH
N_PAD = 10240
RT = N_PAD // NS

def _sc_deg_body(ei_hbm, ones_hbm, z1_hbm, deg_out, deg_sh,
                 dstv, onesv, semA, semB):
    cid = lax.axis_index("c")
    sid = lax.axis_index("s")
    w = cid * NS + sid
    pltpu.sync_copy(z1_hbm, deg_sh.at[pl.ds(sid * RT, RT)])
    pltpu.sync_copy(ones_hbm, onesv)
    pltpu.sync_copy(ei_hbm.at[pl.ds(E + w * EW, EW)], dstv)
    plsc.subcore_barrier()

    def sc(j, sem):
        return pltpu.async_copy(
            onesv, deg_sh.at[dstv.at[pl.ds(j * CH, CH)]], sem, add=True)

    def scw(j, sem):
        pltpu.make_async_copy(
            onesv, deg_sh.at[dstv.at[pl.ds(j * CH, CH)]], sem).wait()

    sc(0, semA)

    def body(i, c):
        j = 2 * i
        sc(j + 1, semB)
        scw(j, semA)

        @pl.when(j + 2 < KF)
        def _():
            sc(j + 2, semA)

        scw(j + 1, semB)
        return c

    lax.fori_loop(0, KF // 2, body, 0)
    pltpu.sync_copy(onesv.at[pl.ds(0, TAIL)],
                    deg_sh.at[dstv.at[pl.ds(KF * CH, TAIL)]], add=True)
    plsc.subcore_barrier()
    pltpu.sync_copy(deg_sh.at[pl.ds(sid * RT, RT)],
                    deg_out.at[cid, pl.ds(sid * RT, RT)])


@functools.cache
def _get_sc_deg():
    mesh = plsc.VectorSubcoreMesh(core_axis_name="c", subcore_axis_name="s")
    return pl.kernel(
        _sc_deg_body,
        out_type=jax.ShapeDtypeStruct((NC, N_PAD), jnp.float32),
        mesh=mesh,
        scratch_types=[
            pltpu.VMEM_SHARED((N_PAD,), jnp.float32),
            pltpu.VMEM((EW,), jnp.int32),
            pltpu.VMEM((CH,), jnp.float32),
            pltpu.SemaphoreType.DMA,
            pltpu.SemaphoreType.DMA,
        ],
    )


def _sc_edge_body(y_hbm, ei_hbm, zr_hbm, acc_out,
                  acc_sh, srcv, dstv0, dstv1, dtail, rows0, rows1,
                  sem0, sem1, semd0, semd1):
    cid = lax.axis_index("c")
    sid = lax.axis_index("s")
    w = cid * NS + sid
    pltpu.sync_copy(zr_hbm, acc_sh.at[pl.ds(sid * RT, RT)])
    pltpu.sync_copy(ei_hbm.at[pl.ds(w * EW, EW)], srcv)
    plsc.subcore_barrier()
    base0 = w * EW

    def g(j, rows, sem):
        return pltpu.async_copy(y_hbm.at[srcv.at[pl.ds(j * CH, CH)]],
                                rows, sem)

    def dld(j, dstv, semd):
        return pltpu.async_copy(ei_hbm.at[pl.ds(E + base0 + j * CH, CH)],
                                dstv, semd)

    def retire(j, rows, sem, dstv, semd):
        pltpu.make_async_copy(y_hbm.at[srcv.at[pl.ds(j * CH, CH)]],
                              rows, sem).wait()
        pltpu.make_async_copy(ei_hbm.at[pl.ds(E + base0 + j * CH, CH)],
                              dstv, semd).wait()
        pltpu.sync_copy(rows, acc_sh.at[dstv], add=True)

    dld(0, dstv0, semd0)
    g(0, rows0, sem0)

    def body(i, c):
        j = 2 * i
        dld(j + 1, dstv1, semd1)
        g(j + 1, rows1, sem1)
        retire(j, rows0, sem0, dstv0, semd0)

        @pl.when(j + 2 < KF)
        def _():
            dld(j + 2, dstv0, semd0)
            g(j + 2, rows0, sem0)

        retire(j + 1, rows1, sem1, dstv1, semd1)
        return c

    lax.fori_loop(0, KF // 2, body, 0)
    pltpu.sync_copy(ei_hbm.at[pl.ds(E + base0 + KF * CH, TAIL)], dtail)
    pltpu.sync_copy(y_hbm.at[srcv.at[pl.ds(KF * CH, TAIL)]],
                    rows0.at[pl.ds(0, TAIL)])
    pltpu.sync_copy(rows0.at[pl.ds(0, TAIL)], acc_sh.at[dtail], add=True)

    plsc.subcore_barrier()
    pltpu.sync_copy(acc_sh.at[pl.ds(sid * RT, RT)],
                    acc_out.at[cid, pl.ds(sid * RT, RT)])


@functools.cache
def _get_sc_edge():
    mesh = plsc.VectorSubcoreMesh(core_axis_name="c", subcore_axis_name="s")
    return pl.kernel(
        _sc_edge_body,
        out_type=jax.ShapeDtypeStruct((NC, N_PAD, D), jnp.float32),
        mesh=mesh,
        scratch_types=[
            pltpu.VMEM_SHARED((N_PAD, D), jnp.float32),
            pltpu.VMEM((EW,), jnp.int32),
            pltpu.VMEM((CH,), jnp.int32),
            pltpu.VMEM((CH,), jnp.int32),
            pltpu.VMEM((TAIL,), jnp.int32),
            pltpu.VMEM((CH, D), jnp.float32),
            pltpu.VMEM((CH, D), jnp.float32),
            pltpu.SemaphoreType.DMA,
            pltpu.SemaphoreType.DMA,
            pltpu.SemaphoreType.DMA,
            pltpu.SemaphoreType.DMA,
        ],
    )


_BN = 1000


def _tc_y1_body(d0_ref, d1_ref, x_ref, w_ref, y_ref, dinv_ref):
    deg = d0_ref[...] + d1_ref[...] + 1.0
    dinv = lax.rsqrt(deg)
    xw = jnp.dot(x_ref[...], w_ref[...], preferred_element_type=jnp.float32)
    y_ref[...] = dinv * xw
    dinv_ref[...] = dinv


def _tc_y1(d0, d1, x, w):
    return pl.pallas_call(
        _tc_y1_body,
        grid=(N // _BN,),
        in_specs=[
            pl.BlockSpec((_BN, 1), lambda i: (i, 0)),
            pl.BlockSpec((_BN, 1), lambda i: (i, 0)),
            pl.BlockSpec((_BN, D), lambda i: (i, 0)),
            pl.BlockSpec((D, D), lambda i: (0, 0)),
        ],
        out_specs=[
            pl.BlockSpec((_BN, D), lambda i: (i, 0)),
            pl.BlockSpec((_BN, 1), lambda i: (i, 0)),
        ],
        out_shape=[
            jax.ShapeDtypeStruct((N, D), jnp.float32),
            jax.ShapeDtypeStruct((N, 1), jnp.float32),
        ],
    )(d0, d1, x, w)


def _tc_mid_body(a_ref, y_ref, dinv_ref, b_ref, w_ref, y2_ref):
    dinv = dinv_ref[...]
    h = b_ref[...] + dinv * (a_ref[0] + a_ref[1] + y_ref[...])
    h = jnp.maximum(h, 0.0)
    y2_ref[...] = dinv * jnp.dot(h, w_ref[...],
                                 preferred_element_type=jnp.float32)


def _tc_mid(accp, y1, dinv, b, w):
    return pl.pallas_call(
        _tc_mid_body,
        grid=(N // _BN,),
        in_specs=[
            pl.BlockSpec((NC, _BN, D), lambda i: (0, i, 0)),
            pl.BlockSpec((_BN, D), lambda i: (i, 0)),
            pl.BlockSpec((_BN, 1), lambda i: (i, 0)),
            pl.BlockSpec((1, D), lambda i: (0, 0)),
            pl.BlockSpec((D, D), lambda i: (0, 0)),
        ],
        out_specs=pl.BlockSpec((_BN, D), lambda i: (i, 0)),
        out_shape=jax.ShapeDtypeStruct((N, D), jnp.float32),
    )(accp, y1, dinv, b, w)


def _tc_final_body(a_ref, y_ref, dinv_ref, b_ref, wfc_ref, bfc_ref,
                   out_ref, g_ref):
    i = pl.program_id(0)
    h = b_ref[...] + dinv_ref[...] * (a_ref[0] + a_ref[1] + y_ref[...])
    h = jnp.maximum(h, 0.0)

    @pl.when(i == 0)
    def _():
        g_ref[...] = jnp.zeros_like(g_ref)

    g_ref[...] += jnp.sum(h, axis=0, keepdims=True)

    @pl.when(i == N // _BN - 1)
    def _():
        g = g_ref[...] * (1.0 / N)
        out_ref[...] = jnp.dot(g, wfc_ref[...],
                               preferred_element_type=jnp.float32) + bfc_ref[...]


def _tc_final(accp, y2, dinv, b, wfc, bfc):
    return pl.pallas_call(
        _tc_final_body,
        grid=(N // _BN,),
        in_specs=[
            pl.BlockSpec((NC, _BN, D), lambda i: (0, i, 0)),
            pl.BlockSpec((_BN, D), lambda i: (i, 0)),
            pl.BlockSpec((_BN, 1), lambda i: (i, 0)),
            pl.BlockSpec((1, D), lambda i: (0, 0)),
            pl.BlockSpec((D, D), lambda i: (0, 0)),
            pl.BlockSpec((1, D), lambda i: (0, 0)),
        ],
        out_specs=pl.BlockSpec((1, D), lambda i: (0, 0)),
        out_shape=jax.ShapeDtypeStruct((1, D), jnp.float32),
        scratch_shapes=[pltpu.VMEM((1, D), jnp.float32)],
    )(accp, y2, dinv, b, wfc, bfc)


def kernel(x, edge_index, W1, b1, W2, b2, Wfc, bfc):
    ei = edge_index.reshape(2 * E)
    ones_ch = jnp.ones((CH,), jnp.float32)
    z1 = jnp.zeros((RT,), jnp.float32)
    zr = jnp.zeros((RT, D), jnp.float32)

    degp = _get_sc_deg()(ei, ones_ch, z1)
    d0 = degp[0, :N].reshape(N, 1)
    d1 = degp[1, :N].reshape(N, 1)

    y1, dinv = _tc_y1(d0, d1, x, W1)

    accp = _get_sc_edge()(y1, ei, zr)
    y2 = _tc_mid(accp, y1, dinv, b1.reshape(1, D), W2)

    accp2 = _get_sc_edge()(y2, ei, zr)
    out = _tc_final(accp2, y2, dinv, b2.reshape(1, D), Wfc, bfc.reshape(1, D))
    return out.reshape(D)
